# baseline (speedup 1.0000x reference)
import functools
import math

import jax
import jax.numpy as jnp
from jax import lax
from jax.experimental import pallas as pl
from jax.experimental.pallas import tpu as pltpu
from jax.experimental.pallas import tpu_sc as plsc

_LANES = 16
_IDX_MINOR = 128
_CHUNK_IDX_ROWS = 5
_CHUNK = _CHUNK_IDX_ROWS * _IDX_MINOR


@functools.lru_cache(maxsize=None)
def _build_sc_gather(n_total: int, vocab: int, d: int):
    info = plsc.get_sparse_core_info()
    nc, ns = info.num_cores, info.num_subcores
    nw = nc * ns
    per_w = n_total // nw
    assert per_w * nw == n_total
    idx_rows = per_w // _IDX_MINOR
    assert idx_rows * _IDX_MINOR == per_w
    n_chunks = idx_rows // _CHUNK_IDX_ROWS
    assert n_chunks * _CHUNK_IDX_ROWS == idx_rows
    scale = math.sqrt(d)
    vregs_per_row = d // _LANES
    rows_per_it = 4
    n_it = _CHUNK // rows_per_it

    mesh = plsc.VectorSubcoreMesh(core_axis_name="c", subcore_axis_name="s")

    @functools.partial(
        pl.kernel,
        mesh=mesh,
        out_type=jax.ShapeDtypeStruct((n_total, d), jnp.float32),
        scratch_types=[
            pltpu.VMEM((idx_rows, _IDX_MINOR), jnp.int32),
            pltpu.VMEM((_CHUNK, d), jnp.float32),
            pltpu.SemaphoreType.DMA,
        ],
        compiler_params=pltpu.CompilerParams(use_tc_tiling_on_sc=False),
    )
    def k(idx_hbm, lut_hbm, out_hbm, idx_v, rows_v, sem):
        wid = lax.axis_index("s") * nc + lax.axis_index("c")
        base = wid * per_w
        pltpu.sync_copy(idx_hbm.at[wid], idx_v)

        def scale_body(i, carry):
            r0 = i * rows_per_it
            for dr in range(rows_per_it):
                for c in range(vregs_per_row):
                    sl = (r0 + dr, pl.ds(c * _LANES, _LANES))
                    rows_v[sl] = rows_v[sl] * scale
            return carry

        for kc in range(n_chunks):
            copies = []
            for j in range(_CHUNK_IDX_ROWS):
                copies.append(pltpu.async_copy(
                    lut_hbm.at[idx_v.at[kc * _CHUNK_IDX_ROWS + j]],
                    rows_v.at[pl.ds(j * _IDX_MINOR, _IDX_MINOR)],
                    sem,
                ))
            for cp in copies:
                cp.wait()
            lax.fori_loop(0, n_it, scale_body, 0)
            pltpu.sync_copy(rows_v, out_hbm.at[pl.ds(base + kc * _CHUNK, _CHUNK)])

    return k


def kernel(input_tokens, lut):
    b, l = input_tokens.shape
    vocab, d = lut.shape
    n_total = b * l
    idx = input_tokens.reshape(-1).astype(jnp.int32)
    info = plsc.get_sparse_core_info()
    nw = info.num_cores * info.num_subcores
    idx3 = idx.reshape(nw, (n_total // nw) // _IDX_MINOR, _IDX_MINOR)
    out = _build_sc_gather(n_total, vocab, d)(idx3, lut)
    return out.reshape(b, l, d)
